# baseline (speedup 1.0000x reference)
import functools

import jax
import jax.numpy as jnp
from jax.experimental import pallas as pl
from jax.experimental.pallas import tpu as pltpu

N = 50000
E = 800000
B = 256
NF = 78
HEADS = 10
OUT = 128
EMB = 320
HP = 16
NB = 98
NBLK = 512
NPAD = NB * NBLK


def _attn_body(x_ref, ss_ref, as1_ref, ad1_ref, xp_ref):
    x = x_ref[...]
    ss = ss_ref[...]
    r = jnp.dot(x, ss, preferred_element_type=jnp.float32)
    as1_ref[...] = r[:, :HP]
    ad1_ref[...] = r[:, HP:]
    xp_ref[...] = jnp.pad(x, ((0, 0), (0, 80 - NF)))


def _attn_vectors(x_pad_rows, ss):
    return pl.pallas_call(
        _attn_body,
        grid=(NB,),
        in_specs=[pl.BlockSpec((NBLK, NF), lambda i: (i, 0)),
                  pl.BlockSpec((NF, 2 * HP), lambda i: (0, 0))],
        out_specs=[pl.BlockSpec((NBLK, HP), lambda i: (i, 0)),
                   pl.BlockSpec((NBLK, HP), lambda i: (i, 0)),
                   pl.BlockSpec((NBLK, 80), lambda i: (i, 0))],
        out_shape=[jax.ShapeDtypeStruct((NPAD, HP), jnp.float32),
                   jax.ShapeDtypeStruct((NPAD, HP), jnp.float32),
                   jax.ShapeDtypeStruct((NPAD, 80), jnp.float32)],
    )(x_pad_rows, ss)


def _proj_body(agg_ref, W1_ref, b1_ref, W2_ref, a2_ref, h2_ref, aa2_ref):
    agg = agg_ref[...]
    outs = []
    for k in range(HEADS):
        ak = agg[:, k * 80:k * 80 + NF]
        wk = W1_ref[:, k * NF:(k + 1) * NF]
        outs.append(jnp.dot(ak, wk, preferred_element_type=jnp.float32))
    h = jnp.concatenate(outs, axis=1) + b1_ref[...]
    h = jnp.where(h > 0, h, jnp.exp(jnp.minimum(h, 0.0)) - 1.0)
    h2 = jnp.dot(h, W2_ref[...], preferred_element_type=jnp.float32)
    h2_ref[...] = h2
    aa2_ref[...] = jnp.dot(h2, a2_ref[...], preferred_element_type=jnp.float32)


def _project(agg1, W1, b1, W2, a2):
    return pl.pallas_call(
        _proj_body,
        grid=(NB,),
        in_specs=[pl.BlockSpec((NBLK, HEADS * 80), lambda i: (i, 0)),
                  pl.BlockSpec((NF, HEADS * NF), lambda i: (0, 0)),
                  pl.BlockSpec((1, HEADS * NF), lambda i: (0, 0)),
                  pl.BlockSpec((HEADS * NF, OUT), lambda i: (0, 0)),
                  pl.BlockSpec((OUT, HP), lambda i: (0, 0))],
        out_specs=[pl.BlockSpec((NBLK, OUT), lambda i: (i, 0)),
                   pl.BlockSpec((NBLK, HP), lambda i: (i, 0))],
        out_shape=[jax.ShapeDtypeStruct((NPAD, OUT), jnp.float32),
                   jax.ShapeDtypeStruct((NPAD, HP), jnp.float32)],
    )(agg1, W1, b1, W2, a2)


def _head_body(xg_ref, te_ref, gW_ref, gb_ref, xtW_ref, xtb_ref, bng_ref,
               bnb_ref, f1a_ref, f1b_ref, f1bias_ref, f2W_ref, f2b_ref,
               oW_ref, ob_ref, out_ref):
    xg = jnp.maximum(jnp.dot(xg_ref[...], gW_ref[...],
                             preferred_element_type=jnp.float32) + gb_ref[...], 0.0)
    xt = jnp.dot(te_ref[...], xtW_ref[...],
                 preferred_element_type=jnp.float32) + xtb_ref[...]
    xt = jnp.maximum(xt * bng_ref[...] + bnb_ref[...], 0.0)
    h1 = jnp.dot(xg, f1a_ref[...], preferred_element_type=jnp.float32)
    h1 = h1 + jnp.dot(xt, f1b_ref[...], preferred_element_type=jnp.float32)
    h1 = jnp.maximum(h1 + f1bias_ref[...], 0.0)
    h2 = jnp.maximum(jnp.dot(h1, f2W_ref[...],
                             preferred_element_type=jnp.float32) + f2b_ref[...], 0.0)
    out_ref[...] = jnp.dot(h2, oW_ref[...],
                           preferred_element_type=jnp.float32) + ob_ref[...]


def _mlp_head(xg_pool, target_embedding, fc_g1_W, fc_g1_b, fc_xt_W, fc_xt_b,
              bn_g, bn_b, fc1_W, fc1_b, fc2_W, fc2_b, out_W, out_b):
    return pl.pallas_call(
        _head_body,
        out_shape=jax.ShapeDtypeStruct((B, 1), jnp.float32),
    )(xg_pool, target_embedding, fc_g1_W, fc_g1_b, fc_xt_W, fc_xt_b,
      bn_g, bn_b, fc1_W[:OUT], fc1_W[OUT:], fc1_b, fc2_W, fc2_b, out_W, out_b)


def kernel(x, edge_index, batch, target_embedding, W1, a_src1, a_dst1, b1,
           W2, a_src2, a_dst2, b2, fc_g1_W, fc_g1_b, fc_xt_W, fc_xt_b,
           bn_g, bn_b, fc1_W, fc1_b, fc2_W, fc2_b, out_W, out_b):
    loop = jnp.arange(N)
    src = jnp.concatenate([edge_index[0], loop])
    dst = jnp.concatenate([edge_index[1], loop])

    Wr = W1.reshape(NF, HEADS, NF)
    vs1 = jnp.einsum('ikj,kj->ik', Wr, a_src1)
    vd1 = jnp.einsum('ikj,kj->ik', Wr, a_dst1)
    ss = jnp.zeros((NF, 2 * HP), jnp.float32)
    ss = ss.at[:, :HEADS].set(vs1).at[:, HP:HP + HEADS].set(vd1)

    xr = jnp.pad(x, ((0, NPAD - N), (0, 0)))
    as1, ad1, xp = _attn_vectors(xr, ss)

    e = jax.nn.leaky_relu(as1[src, :HEADS] + ad1[dst, :HEADS], 0.2)
    w = jnp.exp(e)
    denom = jax.ops.segment_sum(w, dst, num_segments=N)
    coeff = w / denom[dst]
    msg = xp[src][:, None, :] * coeff[:, :, None]
    agg1 = jax.ops.segment_sum(msg.reshape(-1, HEADS * 80), dst, num_segments=N)
    agg1 = jnp.pad(agg1, ((0, NPAD - N), (0, 0)))

    a2 = jnp.zeros((OUT, HP), jnp.float32)
    a2 = a2.at[:, 0].set(a_src2[0]).at[:, 1].set(a_dst2[0])
    h2, aa2 = _project(agg1, W1, b1.reshape(1, -1), W2, a2)

    as2 = aa2[:, 0]
    ad2 = aa2[:, 1]
    e2 = jax.nn.leaky_relu(as2[src] + ad2[dst], 0.2)
    w2 = jnp.exp(e2)
    denom2 = jax.ops.segment_sum(w2, dst, num_segments=N)
    coeff2 = w2 / denom2[dst]
    agg2 = jax.ops.segment_sum(h2[src] * coeff2[:, None], dst, num_segments=N)
    hh = jax.nn.relu(agg2 + b2)

    xg = jax.ops.segment_max(hh, batch, num_segments=B,
                             indices_are_sorted=True)
    xg = jnp.where(jnp.isfinite(xg), xg, 0.0)
    return _mlp_head(xg, target_embedding, fc_g1_W, fc_g1_b, fc_xt_W, fc_xt_b,
                     bn_g, bn_b, fc1_W, fc1_b, fc2_W, fc2_b, out_W, out_b)


# speedup vs baseline: 16.1484x; 16.1484x over previous
import functools

import jax
import jax.numpy as jnp
from jax import lax
from jax.experimental import pallas as pl
from jax.experimental.pallas import tpu as pltpu
from jax.experimental.pallas import tpu_sc as plsc

N = 50000
E = 800000
B = 256
NF = 78
HEADS = 10
OUT = 128
EMB = 320
HP = 16
NB = 98
NBLK = 512
NPAD = NB * NBLK

EPAD = 851968
HALF = EPAD // 2
EPW = EPAD // 32
NG16 = EPW // 16
NCH = EPW // 128
BINW = 64
NBINS = 782
DEADDST = 50048
RPW = 1024

_MESH = plsc.VectorSubcoreMesh(core_axis_name="c", subcore_axis_name="s")


def _leaky(v):
    return jnp.maximum(v, 0.0) + 0.2 * jnp.minimum(v, 0.0)


def _k1_body(src_h, dst_h, bsrc_h, bdst_h, rp_h,
             src_v, dst_v, cnt_v, bs_v, cur_v, allc, pos2d, hist, shc, sem):
    c = lax.axis_index("c")
    s = lax.axis_index("s")
    wid = c * 16 + s
    base_e = wid * EPW
    pltpu.sync_copy(src_h.at[pl.ds(base_e, EPW)], src_v)
    pltpu.sync_copy(dst_h.at[pl.ds(base_e, EPW)], dst_v)

    zero16 = jnp.zeros((16,), jnp.int32)
    iota = lax.iota(jnp.int32, 16)

    def zh(j, _):
        hist[j] = 0
        return 0
    lax.fori_loop(0, RPW, zh, 0)

    def cnt(g, _):
        binv = lax.shift_right_logical(dst_v[pl.ds(g * 16, 16)], 6)
        for i in range(16):
            b = binv[i]
            hist[b] = hist[b] + 1
        return 0
    lax.fori_loop(0, NG16, cnt, 0)

    def pub(j, _):
        acc = zero16
        for i in range(16):
            acc = jnp.where(iota == i, hist[j * 16 + i], acc)
        cnt_v[pl.ds(j * 16, 16)] = acc
        return 0
    lax.fori_loop(0, RPW // 16, pub, 0)

    pltpu.sync_copy(cnt_v, shc.at[s])
    plsc.subcore_barrier()
    for t in range(16):
        pltpu.sync_copy(shc.at[t], allc.at[pl.ds(t * RPW, RPW)])

    def scan(j, tot):
        sl = pl.ds(j * 16, 16)
        t16 = allc[pl.ds(0 * RPW + j * 16, 16)]
        for t in range(1, 16):
            t16 = t16 + allc[pl.ds(t * RPW + j * 16, 16)]
        excl = zero16
        running = tot
        for i in range(16):
            excl = jnp.where(iota == i, running, excl)
            running = running + t16[i]
        bs_v[sl] = excl
        return running
    lax.fori_loop(0, RPW // 16, scan, jnp.int32(0))

    def cur(j, _):
        sl = pl.ds(j * 16, 16)
        acc = bs_v[sl] + c * HALF
        for t in range(16):
            acc = acc + jnp.where(t < s, allc[pl.ds(t * RPW + j * 16, 16)], 0)
        cur_v[sl] = acc
        return 0
    lax.fori_loop(0, RPW // 16, cur, 0)

    @pl.when(s == 0)
    def _():
        pltpu.sync_copy(bs_v, rp_h.at[c])

    def c2s(j, _):
        vec = cur_v[pl.ds(j * 16, 16)]
        for i in range(16):
            hist[j * 16 + i] = vec[i]
        return 0
    lax.fori_loop(0, RPW // 16, c2s, 0)

    def place(g, _):
        binv = lax.shift_right_logical(dst_v[pl.ds(g * 16, 16)], 6)
        posv = zero16
        for i in range(16):
            b = binv[i]
            p = hist[b]
            hist[b] = p + 1
            posv = jnp.where(iota == i, p, posv)
        row = lax.shift_right_logical(g, 3)
        col = lax.bitwise_and(g, 7) * 16
        pos2d[row, pl.ds(col, 16)] = posv
        return 0
    lax.fori_loop(0, NG16, place, 0)

    def scat(j, _):
        sl = pl.ds(j * 128, 128)
        pltpu.async_copy(src_v.at[sl], bsrc_h.at[pos2d.at[j]], sem).wait()
        pltpu.async_copy(dst_v.at[sl], bdst_h.at[pos2d.at[j]], sem).wait()
        return 0
    lax.fori_loop(0, NCH, scat, 0)


def _k1(src_all, dst_all):
    f = pl.kernel(
        _k1_body,
        out_type=[jax.ShapeDtypeStruct((EPAD,), jnp.int32),
                  jax.ShapeDtypeStruct((EPAD,), jnp.int32),
                  jax.ShapeDtypeStruct((2, RPW), jnp.int32)],
        mesh=_MESH,
        scratch_types=[
            pltpu.VMEM((EPW,), jnp.int32),
            pltpu.VMEM((EPW,), jnp.int32),
            pltpu.VMEM((RPW,), jnp.int32),
            pltpu.VMEM((RPW,), jnp.int32),
            pltpu.VMEM((RPW,), jnp.int32),
            pltpu.VMEM((16 * RPW,), jnp.int32),
            pltpu.VMEM((NCH, 128), jnp.int32),
            pltpu.SMEM((RPW,), jnp.int32),
            pltpu.VMEM_SHARED((16, RPW), jnp.int32),
            pltpu.SemaphoreType.DMA,
        ],
    )
    return f(src_all, dst_all)


def _k3_body(bsrc_h, bdst_h, rp_h, comb_h, agg_h,
             rp_v, adl, accum, denom, rden, srcs, dsts, cg, sem):
    c = lax.axis_index("c")
    s = lax.axis_index("s")
    wid = c * 16 + s
    for r in range(2):
        pltpu.sync_copy(rp_h.at[r], rp_v.at[pl.ds(r * RPW, RPW)])
    zf16 = jnp.zeros((16,), jnp.float32)
    iota = lax.iota(jnp.int32, 16)

    def bin_iter(t, _):
        b = wid + 32 * t

        @pl.when(b < NBINS)
        def _():
            base_node = b * BINW

            def zacc(j, _):
                for q in range(50):
                    accum[j, pl.ds(q * 16, 16)] = zf16
                return 0
            lax.fori_loop(0, BINW + 1, zacc, 0)

            def zden(j, _):
                denom[pl.ds(j * 16, 16)] = zf16
                return 0
            lax.fori_loop(0, BINW + 1, zden, 0)

            pltpu.sync_copy(comb_h.at[pl.ds(base_node, BINW)],
                            adl.at[pl.ds(0, BINW)])

            def make_region(pass2):
                def do_region(r, _):
                    se = rp_v[pl.ds(r * RPW + b, 16)]
                    start = se[0]
                    end = se[1]
                    s0 = lax.bitwise_and(start, -8)
                    glob0 = r * HALF + s0
                    nch = lax.shift_right_logical(end - s0 + 127, 7)

                    def chunk(j, _):
                        off = pl.multiple_of(glob0 + j * 128, 8)
                        pltpu.sync_copy(bsrc_h.at[pl.ds(off, 128)], srcs)
                        pltpu.sync_copy(bdst_h.at[pl.ds(off, 128)], dsts)
                        pltpu.async_copy(comb_h.at[srcs], cg, sem).wait()

                        def grp(g, _):
                            dv = dsts[pl.ds(g * 16, 16)]
                            egv = s0 + j * 128 + g * 16 + iota
                            valid = jnp.logical_and(egv >= start, egv < end)
                            dlv = jnp.where(valid, dv - base_node, BINW)
                            for i in range(16):
                                dl = dlv[i]
                                asrow = cg[g * 16 + i, pl.ds(0, 16)]
                                w = jnp.exp(_leaky(
                                    asrow + adl[dl, pl.ds(16, 16)]))
                                if not pass2:
                                    plsc.addupdate(
                                        denom.at[pl.ds(dl * 16, 16)], w)
                                else:
                                    coeff = w * rden[pl.ds(dl * 16, 16)]
                                    for k in range(HEADS):
                                        ck = coeff[k]
                                        for q in range(5):
                                            plsc.addupdate(
                                                accum.at[dl, pl.ds(
                                                    k * 80 + q * 16, 16)],
                                                ck * cg[g * 16 + i,
                                                        pl.ds(32 + q * 16, 16)])
                            return 0
                        lax.fori_loop(0, 8, grp, 0)
                        return 0
                    lax.fori_loop(0, nch, chunk, 0)
                    return 0
                return do_region

            lax.fori_loop(0, 2, make_region(False), 0)

            def recip(j, _):
                rden[pl.ds(j * 16, 16)] = 1.0 / denom[pl.ds(j * 16, 16)]
                return 0
            lax.fori_loop(0, BINW + 1, recip, 0)

            lax.fori_loop(0, 2, make_region(True), 0)

            pltpu.sync_copy(accum.at[pl.ds(0, BINW)],
                            agg_h.at[pl.ds(base_node, BINW)])
        return 0
    lax.fori_loop(0, 25, bin_iter, 0)


def _k3(bsrc, bdst, rp, comb):
    f = pl.kernel(
        _k3_body,
        out_type=[jax.ShapeDtypeStruct((NPAD, HEADS * 80), jnp.float32)],
        mesh=_MESH,
        scratch_types=[
            pltpu.VMEM((2 * RPW,), jnp.int32),
            pltpu.VMEM((BINW + 1, 128), jnp.float32),
            pltpu.VMEM((BINW + 1, 800), jnp.float32),
            pltpu.VMEM(((BINW + 1) * HP,), jnp.float32),
            pltpu.VMEM(((BINW + 1) * HP,), jnp.float32),
            pltpu.VMEM((128,), jnp.int32),
            pltpu.VMEM((128,), jnp.int32),
            pltpu.VMEM((128, 128), jnp.float32),
            pltpu.SemaphoreType.DMA,
        ],
    )
    return f(bsrc, bdst, rp, comb)[0]


def _k5_body(bsrc_h, bdst_h, rp_h, as2_h, ad2_h, h2_h, b2_h, hh_h,
             rp_v, accum, denom, rden, srcs, dsts, sag, dag, h2g, b2v, sem):
    c = lax.axis_index("c")
    s = lax.axis_index("s")
    wid = c * 16 + s
    for r in range(2):
        pltpu.sync_copy(rp_h.at[r], rp_v.at[pl.ds(r * RPW, RPW)])
    pltpu.sync_copy(b2_h, b2v)
    zf16 = jnp.zeros((16,), jnp.float32)
    iota = lax.iota(jnp.int32, 16)

    def bin_iter(t, _):
        b = wid + 32 * t

        @pl.when(b < NBINS)
        def _():
            base_node = b * BINW

            def zacc(j, _):
                for q in range(8):
                    accum[j, pl.ds(q * 16, 16)] = zf16
                return 0
            lax.fori_loop(0, BINW + 1, zacc, 0)

            def zden(j, _):
                denom[pl.ds(j * 16, 16)] = zf16
                return 0
            lax.fori_loop(0, BINW + 1, zden, 0)

            def make_region(pass2):
                def do_region(r, _):
                    se = rp_v[pl.ds(r * RPW + b, 16)]
                    start = se[0]
                    end = se[1]
                    s0 = lax.bitwise_and(start, -8)
                    glob0 = r * HALF + s0
                    nch = lax.shift_right_logical(end - s0 + 127, 7)

                    def chunk(j, _):
                        off = pl.multiple_of(glob0 + j * 128, 8)
                        pltpu.sync_copy(bsrc_h.at[pl.ds(off, 128)], srcs)
                        pltpu.sync_copy(bdst_h.at[pl.ds(off, 128)], dsts)
                        pltpu.async_copy(as2_h.at[srcs], sag, sem).wait()
                        pltpu.async_copy(ad2_h.at[dsts], dag, sem).wait()
                        if pass2:
                            pltpu.async_copy(h2_h.at[srcs], h2g, sem).wait()

                        def grp(g, _):
                            dv = dsts[pl.ds(g * 16, 16)]
                            egv = s0 + j * 128 + g * 16 + iota
                            valid = jnp.logical_and(egv >= start, egv < end)
                            dlv = jnp.where(valid, dv - base_node, BINW)
                            sa = sag[pl.ds(g * 16, 16)]
                            da = dag[pl.ds(g * 16, 16)]
                            w = jnp.exp(_leaky(sa + da))
                            if not pass2:
                                for i in range(16):
                                    wspl = jnp.full((16,), w[i], jnp.float32)
                                    plsc.addupdate(
                                        denom.at[pl.ds(dlv[i] * 16, 16)],
                                        wspl)
                            else:
                                for i in range(16):
                                    dl = dlv[i]
                                    rv = rden[pl.ds(dl * 16, 16)]
                                    ck = w[i] * rv[0]
                                    for q in range(8):
                                        plsc.addupdate(
                                            accum.at[dl, pl.ds(q * 16, 16)],
                                            ck * h2g[g * 16 + i,
                                                     pl.ds(q * 16, 16)])
                            return 0
                        lax.fori_loop(0, 8, grp, 0)
                        return 0
                    lax.fori_loop(0, nch, chunk, 0)
                    return 0
                return do_region

            lax.fori_loop(0, 2, make_region(False), 0)

            def recip(j, _):
                rden[pl.ds(j * 16, 16)] = 1.0 / denom[pl.ds(j * 16, 16)]
                return 0
            lax.fori_loop(0, BINW + 1, recip, 0)

            lax.fori_loop(0, 2, make_region(True), 0)

            def act(row, _):
                for q in range(8):
                    sl = pl.ds(q * 16, 16)
                    accum[row, sl] = jnp.maximum(
                        accum[row, sl] + b2v[sl], 0.0)
                return 0
            lax.fori_loop(0, BINW, act, 0)

            pltpu.sync_copy(accum.at[pl.ds(0, BINW)],
                            hh_h.at[pl.ds(base_node, BINW)])
        return 0
    lax.fori_loop(0, 25, bin_iter, 0)


def _k5(bsrc, bdst, rp, as2, ad2, h2, b2):
    f = pl.kernel(
        _k5_body,
        out_type=[jax.ShapeDtypeStruct((NPAD, OUT), jnp.float32)],
        mesh=_MESH,
        scratch_types=[
            pltpu.VMEM((2 * RPW,), jnp.int32),
            pltpu.VMEM((BINW + 1, OUT), jnp.float32),
            pltpu.VMEM(((BINW + 1) * 16,), jnp.float32),
            pltpu.VMEM(((BINW + 1) * 16,), jnp.float32),
            pltpu.VMEM((128,), jnp.int32),
            pltpu.VMEM((128,), jnp.int32),
            pltpu.VMEM((128,), jnp.float32),
            pltpu.VMEM((128,), jnp.float32),
            pltpu.VMEM((128, OUT), jnp.float32),
            pltpu.VMEM((OUT,), jnp.float32),
            pltpu.SemaphoreType.DMA,
        ],
    )
    return f(bsrc, bdst, rp, as2, ad2, h2, b2)[0]


def _attn_body(x_ref, ss_ref, comb_ref):
    x = x_ref[...]
    r = jnp.dot(x, ss_ref[...], preferred_element_type=jnp.float32)
    comb_ref[...] = jnp.concatenate(
        [r, jnp.pad(x, ((0, 0), (0, 96 - NF)))], axis=1)


def _attn_vectors(x_pad_rows, ss):
    return pl.pallas_call(
        _attn_body,
        grid=(NB,),
        in_specs=[pl.BlockSpec((NBLK, NF), lambda i: (i, 0)),
                  pl.BlockSpec((NF, 2 * HP), lambda i: (0, 0))],
        out_specs=pl.BlockSpec((NBLK, 128), lambda i: (i, 0)),
        out_shape=jax.ShapeDtypeStruct((NPAD, 128), jnp.float32),
    )(x_pad_rows, ss)


def _proj_body(agg_ref, W1_ref, b1_ref, W2_ref, a2_ref, h2_ref, aa2_ref):
    agg = agg_ref[...]
    outs = []
    for k in range(HEADS):
        ak = agg[:, k * 80:k * 80 + NF]
        wk = W1_ref[:, k * NF:(k + 1) * NF]
        outs.append(jnp.dot(ak, wk, preferred_element_type=jnp.float32))
    h = jnp.concatenate(outs, axis=1) + b1_ref[...]
    h = jnp.where(h > 0, h, jnp.exp(jnp.minimum(h, 0.0)) - 1.0)
    h2 = jnp.dot(h, W2_ref[...], preferred_element_type=jnp.float32)
    h2_ref[...] = h2
    aa2_ref[...] = jnp.dot(h2, a2_ref[...], preferred_element_type=jnp.float32)


def _project(agg1, W1, b1, W2, a2):
    return pl.pallas_call(
        _proj_body,
        grid=(NB,),
        in_specs=[pl.BlockSpec((NBLK, HEADS * 80), lambda i: (i, 0)),
                  pl.BlockSpec((NF, HEADS * NF), lambda i: (0, 0)),
                  pl.BlockSpec((1, HEADS * NF), lambda i: (0, 0)),
                  pl.BlockSpec((HEADS * NF, OUT), lambda i: (0, 0)),
                  pl.BlockSpec((OUT, HP), lambda i: (0, 0))],
        out_specs=[pl.BlockSpec((NBLK, OUT), lambda i: (i, 0)),
                   pl.BlockSpec((NBLK, HP), lambda i: (i, 0))],
        out_shape=[jax.ShapeDtypeStruct((NPAD, OUT), jnp.float32),
                   jax.ShapeDtypeStruct((NPAD, HP), jnp.float32)],
    )(agg1, W1, b1, W2, a2)


def _head_body(xg_ref, te_ref, gW_ref, gb_ref, xtW_ref, xtb_ref, bng_ref,
               bnb_ref, f1a_ref, f1b_ref, f1bias_ref, f2W_ref, f2b_ref,
               oW_ref, ob_ref, out_ref):
    xg = jnp.maximum(jnp.dot(xg_ref[...], gW_ref[...],
                             preferred_element_type=jnp.float32) + gb_ref[...], 0.0)
    xt = jnp.dot(te_ref[...], xtW_ref[...],
                 preferred_element_type=jnp.float32) + xtb_ref[...]
    xt = jnp.maximum(xt * bng_ref[...] + bnb_ref[...], 0.0)
    h1 = jnp.dot(xg, f1a_ref[...], preferred_element_type=jnp.float32)
    h1 = h1 + jnp.dot(xt, f1b_ref[...], preferred_element_type=jnp.float32)
    h1 = jnp.maximum(h1 + f1bias_ref[...], 0.0)
    h2 = jnp.maximum(jnp.dot(h1, f2W_ref[...],
                             preferred_element_type=jnp.float32) + f2b_ref[...], 0.0)
    out_ref[...] = jnp.dot(h2, oW_ref[...],
                           preferred_element_type=jnp.float32) + ob_ref[...]


def _mlp_head(xg_pool, target_embedding, fc_g1_W, fc_g1_b, fc_xt_W, fc_xt_b,
              bn_g, bn_b, fc1_W, fc1_b, fc2_W, fc2_b, out_W, out_b):
    return pl.pallas_call(
        _head_body,
        out_shape=jax.ShapeDtypeStruct((B, 1), jnp.float32),
    )(xg_pool, target_embedding, fc_g1_W, fc_g1_b, fc_xt_W, fc_xt_b,
      bn_g, bn_b, fc1_W[:OUT], fc1_W[OUT:], fc1_b, fc2_W, fc2_b, out_W, out_b)


def kernel(x, edge_index, batch, target_embedding, W1, a_src1, a_dst1, b1,
           W2, a_src2, a_dst2, b2, fc_g1_W, fc_g1_b, fc_xt_W, fc_xt_b,
           bn_g, bn_b, fc1_W, fc1_b, fc2_W, fc2_b, out_W, out_b):
    loop = jnp.arange(N, dtype=jnp.int32)
    npad = EPAD - E - N
    src_all = jnp.concatenate([edge_index[0].astype(jnp.int32), loop,
                               jnp.zeros((npad,), jnp.int32)])
    dst_all = jnp.concatenate([edge_index[1].astype(jnp.int32), loop,
                               jnp.full((npad,), DEADDST, jnp.int32)])

    bsrc, bdst, rp = _k1(src_all, dst_all)

    Wr = W1.reshape(NF, HEADS, NF)
    vs1 = jnp.einsum('ikj,kj->ik', Wr, a_src1)
    vd1 = jnp.einsum('ikj,kj->ik', Wr, a_dst1)
    ss = jnp.zeros((NF, 2 * HP), jnp.float32)
    ss = ss.at[:, :HEADS].set(vs1).at[:, HP:HP + HEADS].set(vd1)

    xr = jnp.pad(x, ((0, NPAD - N), (0, 0)))
    comb = _attn_vectors(xr, ss)

    agg1 = _k3(bsrc, bdst, rp, comb)

    a2 = jnp.zeros((OUT, HP), jnp.float32)
    a2 = a2.at[:, 0].set(a_src2[0]).at[:, 1].set(a_dst2[0])
    h2, aa2 = _project(agg1, W1, b1.reshape(1, -1), W2, a2)

    as2 = aa2[:, 0]
    ad2 = aa2[:, 1]
    hh = _k5(bsrc, bdst, rp, as2, ad2, h2, b2)

    xg = jax.ops.segment_max(hh[:N], batch, num_segments=B,
                             indices_are_sorted=True)
    xg = jnp.where(jnp.isfinite(xg), xg, 0.0)
    return _mlp_head(xg, target_embedding, fc_g1_W, fc_g1_b, fc_xt_W, fc_xt_b,
                     bn_g, bn_b, fc1_W, fc1_b, fc2_W, fc2_b, out_W, out_b)
